# Initial kernel scaffold; baseline (speedup 1.0000x reference)
import jax
import jax.numpy as jnp
from jax.experimental import pallas as pl


def kernel(x, w16, b1, w2, b2, wih, whh, gbias, lw, lb):
    raise NotImplementedError("write your pallas kernel here")


# speedup vs baseline: 10.5225x; 10.5225x over previous
import functools

import jax
import jax.numpy as jnp
from jax import lax
from jax.experimental import pallas as pl
from jax.experimental.pallas import tpu as pltpu

C1 = 64
C2 = 32
KH = 3
NPH = 18


def _make_conv_kernel(bb, T, TP):
    K1 = KH * C1

    def body(xph_ref, w16_ref, b1_ref, w2c_ref, b2_ref, wih_ref, gbt_ref,
             out_ref, h1_ref):
        lane = lax.broadcasted_iota(jnp.int32, (C1, TP), 1)
        b1 = b1_ref[...]
        b2 = b2_ref[...]
        gbt = gbt_ref[...]
        w2c = w2c_ref[...]
        wih = wih_ref[...]
        zcol = jnp.zeros((C1, 1), jnp.float32)

        for b in range(bb):
            for kh in range(KH):
                r1 = jnp.dot(w16_ref[...], xph_ref[b, kh],
                             preferred_element_type=jnp.float32)
                pp = r1.reshape(4, 4, C1, TP)
                pooled = jnp.maximum(jnp.maximum(pp[:, 0], pp[:, 1]),
                                     jnp.maximum(pp[:, 2], pp[:, 3]))
                ph = [pooled[d] + b1 for d in range(4)]
                p0 = jnp.where(lane < T, ph[0], 0.0)
                row = kh * C1
                h1_ref[1 * K1 + row:1 * K1 + row + C1, :] = p0
                h1_ref[2 * K1 + row:2 * K1 + row + C1, :] = ph[1]
                h1_ref[3 * K1 + row:3 * K1 + row + C1, :] = ph[2]
                h1_ref[4 * K1 + row:4 * K1 + row + C1, :] = ph[3]
                h1_ref[0 * K1 + row:0 * K1 + row + C1, :] = jnp.concatenate(
                    [zcol, ph[3][:, :TP - 1]], axis=1)
                h1_ref[5 * K1 + row:5 * K1 + row + C1, :] = jnp.concatenate(
                    [p0[:, 1:], zcol], axis=1)

            f0 = jnp.dot(w2c, h1_ref[0 * K1:3 * K1, :],
                         preferred_element_type=jnp.float32)
            f1 = jnp.dot(w2c, h1_ref[1 * K1:4 * K1, :],
                         preferred_element_type=jnp.float32)
            f2 = jnp.dot(w2c, h1_ref[2 * K1:5 * K1, :],
                         preferred_element_type=jnp.float32)
            f3 = jnp.dot(w2c, h1_ref[3 * K1:6 * K1, :],
                         preferred_element_type=jnp.float32)
            feat = jnp.maximum(jnp.maximum(f0, f1),
                               jnp.maximum(f2, f3)) + b2

            proj = lax.dot_general(
                wih, feat, dimension_numbers=(((0,), (0,)), ((), ())),
                preferred_element_type=jnp.float32)
            out_ref[b] = proj + gbt

    return body


def _make_scan_kernel(T):
    def body(ga_ref, whh_ref, lw_ref, out_ref):
        whh = whh_ref[...]

        def step(t, carry):
            h, c, acc = carry
            gt = ga_ref[t]
            ti = jnp.tanh(gt[0] + h * whh[0])
            tf = jnp.tanh(gt[1] + h * whh[1])
            tg = jnp.tanh(gt[2] + h * whh[2])
            to = jnp.tanh(gt[3] + h * whh[3])
            i_s = ti * 0.5 + 0.5
            f_s = tf * 0.5 + 0.5
            o_s = to * 0.5 + 0.5
            c = f_s * c + i_s * tg
            h = o_s * jnp.tanh(c)
            acc = acc + h * lw_ref[t]
            return h, c, acc

        z = jnp.zeros((8, 128), jnp.float32)
        _, _, acc = lax.fori_loop(0, T, step, (z, z, z))
        out_ref[...] = acc

    return body


@jax.jit
def _run(x, w16, b1, w2, b2, wih, whh, gbias, lw, lb):
    B, H, W = x.shape
    T = lw.shape[0]
    TP = ((T + 127) // 128) * 128
    bb = 4
    assert H == KH and W == 16 * T and B % bb == 0 and B % 512 == 0

    xpad = jnp.pad(x, ((0, 0), (0, 0), (1, 1)))
    xph = jnp.stack([xpad[:, :, n::16][:, :, :T] for n in range(NPH)], axis=2)
    xph = jnp.pad(xph, ((0, 0), (0, 0), (0, 0), (0, TP - T)))

    w2c = w2[0:C2, 0:3 * KH * C1]

    conv_body = _make_conv_kernel(bb, T, TP)
    pa = pl.pallas_call(
        conv_body,
        out_shape=jax.ShapeDtypeStruct((B, 8, TP), jnp.float32),
        grid=(B // bb,),
        in_specs=[
            pl.BlockSpec((bb, KH, NPH, TP), lambda g: (g, 0, 0, 0)),
            pl.BlockSpec((16 * C1, NPH), lambda g: (0, 0)),
            pl.BlockSpec((C1, 1), lambda g: (0, 0)),
            pl.BlockSpec((C2, 3 * KH * C1), lambda g: (0, 0)),
            pl.BlockSpec((C2, 1), lambda g: (0, 0)),
            pl.BlockSpec((C1 // 2, 8), lambda g: (0, 0)),
            pl.BlockSpec((8, 1), lambda g: (0, 0)),
        ],
        out_specs=pl.BlockSpec((bb, 8, TP), lambda g: (g, 0, 0)),
        scratch_shapes=[pltpu.VMEM((6 * KH * C1, TP), jnp.float32)],
        compiler_params=pltpu.CompilerParams(
            dimension_semantics=("parallel",)),
    )(xph, w16, b1, w2c, b2, wih, gbias.T)

    pat = pa[:, :, :T].reshape(4, 128, 2, 4, T)
    fwd = pat[:, :, 0]
    bwd = pat[:, :, 1][..., ::-1]
    ga = jnp.stack([fwd, bwd], axis=0)
    ga = ga.transpose(4, 3, 0, 1, 2).reshape(T, 4, 8, 128)

    whh_arr = jnp.repeat(whh, 4, axis=0).T.reshape(4, 8, 1)
    lw_arr = jnp.concatenate(
        [jnp.tile(lw[:, 0:1], (1, 4)), jnp.tile(lw[::-1, 1:2], (1, 4))],
        axis=1).reshape(T, 8, 1)

    scan_body = _make_scan_kernel(T)
    acc = pl.pallas_call(
        scan_body,
        out_shape=jax.ShapeDtypeStruct((8, 128), jnp.float32),
    )(ga, whh_arr, lw_arr)

    return (acc[0:4] + acc[4:8] + lb[0, 0]).reshape(B)


def kernel(x, w16, b1, w2, b2, wih, whh, gbias, lw, lb):
    return _run(x, w16, b1, w2, b2, wih, whh, gbias, lw, lb)


# speedup vs baseline: 17.9890x; 1.7096x over previous
import functools

import jax
import jax.numpy as jnp
from jax import lax
from jax.experimental import pallas as pl
from jax.experimental.pallas import tpu as pltpu

C1 = 64
C2 = 32
KH = 3
NPH = 18


def _make_conv_kernel(bb, T, TP):
    K1 = KH * C1

    def body(xph_ref, w16_ref, b1_ref, w2c_ref, b2_ref, wih_ref, gbt_ref,
             out_ref, h1_ref):
        lane = lax.broadcasted_iota(jnp.int32, (C1, TP), 1)
        b1 = b1_ref[...]
        b2 = b2_ref[...]
        gbt = gbt_ref[...]
        w2c = w2c_ref[...]
        wih = wih_ref[...]
        zcol = jnp.zeros((C1, 1), jnp.float32)

        for b in range(bb):
            for kh in range(KH):
                r1 = jnp.dot(w16_ref[...], xph_ref[b, kh],
                             preferred_element_type=jnp.float32)
                pp = r1.reshape(4, 4, C1, TP)
                pooled = jnp.maximum(jnp.maximum(pp[:, 0], pp[:, 1]),
                                     jnp.maximum(pp[:, 2], pp[:, 3]))
                ph = [pooled[d] + b1 for d in range(4)]
                p0 = jnp.where(lane < T, ph[0], 0.0)
                row = kh * C1
                h1_ref[1 * K1 + row:1 * K1 + row + C1, :] = p0
                h1_ref[2 * K1 + row:2 * K1 + row + C1, :] = ph[1]
                h1_ref[3 * K1 + row:3 * K1 + row + C1, :] = ph[2]
                h1_ref[4 * K1 + row:4 * K1 + row + C1, :] = ph[3]
                h1_ref[0 * K1 + row:0 * K1 + row + C1, :] = jnp.concatenate(
                    [zcol, ph[3][:, :TP - 1]], axis=1)
                h1_ref[5 * K1 + row:5 * K1 + row + C1, :] = jnp.concatenate(
                    [p0[:, 1:], zcol], axis=1)

            f0 = jnp.dot(w2c, h1_ref[0 * K1:3 * K1, :],
                         preferred_element_type=jnp.float32)
            f1 = jnp.dot(w2c, h1_ref[1 * K1:4 * K1, :],
                         preferred_element_type=jnp.float32)
            f2 = jnp.dot(w2c, h1_ref[2 * K1:5 * K1, :],
                         preferred_element_type=jnp.float32)
            f3 = jnp.dot(w2c, h1_ref[3 * K1:6 * K1, :],
                         preferred_element_type=jnp.float32)
            feat = jnp.maximum(jnp.maximum(f0, f1),
                               jnp.maximum(f2, f3)) + b2

            proj = lax.dot_general(
                wih, feat, dimension_numbers=(((0,), (0,)), ((), ())),
                preferred_element_type=jnp.float32)
            out_ref[b] = proj + gbt

    return body


def _make_scan_kernel(T):
    def body(ga_ref, whh_ref, lw_ref, out_ref):
        whh = whh_ref[...]

        def step(t, carry):
            h, c, acc = carry
            gt = ga_ref[t]
            ti = jnp.tanh(gt[0] + h * whh[0])
            tf = jnp.tanh(gt[1] + h * whh[1])
            tg = jnp.tanh(gt[2] + h * whh[2])
            to = jnp.tanh(gt[3] + h * whh[3])
            i_s = ti * 0.5 + 0.5
            f_s = tf * 0.5 + 0.5
            o_s = to * 0.5 + 0.5
            c = f_s * c + i_s * tg
            h = o_s * jnp.tanh(c)
            acc = acc + h * lw_ref[t]
            return h, c, acc

        z = jnp.zeros((8, 128), jnp.float32)
        _, _, acc = lax.fori_loop(0, T, step, (z, z, z))
        out_ref[...] = acc

    return body


@jax.jit
def _run(x, w16, b1, w2, b2, wih, whh, gbias, lw, lb):
    B, H, W = x.shape
    T = lw.shape[0]
    TP = ((T + 127) // 128) * 128
    bb = 4
    assert H == KH and W == 16 * T and B % bb == 0 and B % 512 == 0

    xpad = jnp.pad(x, ((0, 0), (0, 0), (1, 16 * (T + 1) - W - 1)))
    x16 = xpad.reshape(B, KH, T + 1, 16).transpose(0, 1, 3, 2)
    xph = jnp.concatenate([x16[:, :, :, :T], x16[:, :, 0:2, 1:T + 1]], axis=2)
    xph = jnp.pad(xph, ((0, 0), (0, 0), (0, 0), (0, TP - T)))

    w2c = w2[0:C2, 0:3 * KH * C1]

    conv_body = _make_conv_kernel(bb, T, TP)
    pa = pl.pallas_call(
        conv_body,
        out_shape=jax.ShapeDtypeStruct((B, 8, TP), jnp.float32),
        grid=(B // bb,),
        in_specs=[
            pl.BlockSpec((bb, KH, NPH, TP), lambda g: (g, 0, 0, 0)),
            pl.BlockSpec((16 * C1, NPH), lambda g: (0, 0)),
            pl.BlockSpec((C1, 1), lambda g: (0, 0)),
            pl.BlockSpec((C2, 3 * KH * C1), lambda g: (0, 0)),
            pl.BlockSpec((C2, 1), lambda g: (0, 0)),
            pl.BlockSpec((C1 // 2, 8), lambda g: (0, 0)),
            pl.BlockSpec((8, 1), lambda g: (0, 0)),
        ],
        out_specs=pl.BlockSpec((bb, 8, TP), lambda g: (g, 0, 0)),
        scratch_shapes=[pltpu.VMEM((6 * KH * C1, TP), jnp.float32)],
        compiler_params=pltpu.CompilerParams(
            dimension_semantics=("parallel",)),
    )(xph, w16, b1, w2c, b2, wih, gbias.T)

    pat = pa[:, :, :T].reshape(4, 128, 2, 4, T)
    fwd = pat[:, :, 0]
    bwd = pat[:, :, 1][..., ::-1]
    ga = jnp.stack([fwd, bwd], axis=0)
    ga = ga.transpose(4, 3, 0, 1, 2).reshape(T, 4, 8, 128)

    whh_arr = jnp.repeat(whh, 4, axis=0).T.reshape(4, 8, 1)
    lw_arr = jnp.concatenate(
        [jnp.tile(lw[:, 0:1], (1, 4)), jnp.tile(lw[::-1, 1:2], (1, 4))],
        axis=1).reshape(T, 8, 1)

    scan_body = _make_scan_kernel(T)
    acc = pl.pallas_call(
        scan_body,
        out_shape=jax.ShapeDtypeStruct((8, 128), jnp.float32),
    )(ga, whh_arr, lw_arr)

    return (acc[0:4] + acc[4:8] + lb[0, 0]).reshape(B)


def kernel(x, w16, b1, w2, b2, wih, whh, gbias, lw, lb):
    return _run(x, w16, b1, w2, b2, wih, whh, gbias, lw, lb)


# speedup vs baseline: 18.5048x; 1.0287x over previous
import functools

import jax
import jax.numpy as jnp
from jax import lax
from jax.experimental import pallas as pl
from jax.experimental.pallas import tpu as pltpu

C1 = 64
C2 = 32
KH = 3
NPH = 18


def _make_conv_kernel(bb, T, TP):
    K1 = KH * C1

    def body(xph_ref, w16_ref, b1_ref, w2c_ref, b2_ref, wih_ref, gbt_ref,
             out_ref, h1_ref):
        lane = lax.broadcasted_iota(jnp.int32, (C1, TP), 1)
        b1 = b1_ref[...]
        b2 = b2_ref[...]
        gbt = gbt_ref[...]
        w2c = w2c_ref[...]
        wih = wih_ref[...]
        zcol = jnp.zeros((C1, 1), jnp.bfloat16)

        for b in range(bb):
            for kh in range(KH):
                ph = []
                for d in range(4):
                    rd = jnp.dot(w16_ref[256 * d:256 * d + 256, :],
                                 xph_ref[b, kh],
                                 preferred_element_type=jnp.float32)
                    pp = rd.reshape(4, C1, TP)
                    ph.append(((jnp.maximum(jnp.maximum(pp[0], pp[1]),
                                            jnp.maximum(pp[2], pp[3])) + b1)
                               ).astype(jnp.bfloat16))
                p0 = jnp.where(lane < T, ph[0], 0.0)
                row = kh * C1
                h1_ref[1 * K1 + row:1 * K1 + row + C1, :] = p0
                h1_ref[2 * K1 + row:2 * K1 + row + C1, :] = ph[1]
                h1_ref[3 * K1 + row:3 * K1 + row + C1, :] = ph[2]
                h1_ref[4 * K1 + row:4 * K1 + row + C1, :] = ph[3]
                h1_ref[0 * K1 + row:0 * K1 + row + C1, :] = jnp.concatenate(
                    [zcol, ph[3][:, :TP - 1]], axis=1)
                h1_ref[5 * K1 + row:5 * K1 + row + C1, :] = jnp.concatenate(
                    [p0[:, 1:], zcol], axis=1)

            f0 = jnp.dot(w2c, h1_ref[0 * K1:3 * K1, :],
                         preferred_element_type=jnp.float32)
            f1 = jnp.dot(w2c, h1_ref[1 * K1:4 * K1, :],
                         preferred_element_type=jnp.float32)
            f2 = jnp.dot(w2c, h1_ref[2 * K1:5 * K1, :],
                         preferred_element_type=jnp.float32)
            f3 = jnp.dot(w2c, h1_ref[3 * K1:6 * K1, :],
                         preferred_element_type=jnp.float32)
            feat = jnp.maximum(jnp.maximum(f0, f1),
                               jnp.maximum(f2, f3)) + b2

            proj = lax.dot_general(
                wih, feat, dimension_numbers=(((0,), (0,)), ((), ())),
                preferred_element_type=jnp.float32)
            out_ref[b] = proj + gbt

    return body


def _make_scan_kernel(T):
    def body(ga_ref, whh_ref, lw_ref, out_ref):
        whh = whh_ref[...]

        def step(t, carry):
            h, c, acc = carry
            gt = ga_ref[t]
            ti = jnp.tanh(gt[0] + h * whh[0])
            tf = jnp.tanh(gt[1] + h * whh[1])
            tg = jnp.tanh(gt[2] + h * whh[2])
            to = jnp.tanh(gt[3] + h * whh[3])
            i_s = ti * 0.5 + 0.5
            f_s = tf * 0.5 + 0.5
            o_s = to * 0.5 + 0.5
            c = f_s * c + i_s * tg
            h = o_s * jnp.tanh(c)
            acc = acc + h * lw_ref[t]
            return h, c, acc

        z = jnp.zeros((8, 128), jnp.float32)
        _, _, acc = lax.fori_loop(0, T, step, (z, z, z))
        out_ref[...] = acc

    return body


@jax.jit
def _run(x, w16, b1, w2, b2, wih, whh, gbias, lw, lb):
    B, H, W = x.shape
    T = lw.shape[0]
    TP = ((T + 127) // 128) * 128
    bb = 4
    assert H == KH and W == 16 * T and B % bb == 0 and B % 512 == 0

    xpad = jnp.pad(x.astype(jnp.bfloat16),
                   ((0, 0), (0, 0), (1, 16 * (T + 1) - W - 1)))
    x16 = xpad.reshape(B, KH, T + 1, 16).transpose(0, 1, 3, 2)
    xph = jnp.concatenate([x16[:, :, :, :T], x16[:, :, 0:2, 1:T + 1]], axis=2)
    xph = jnp.pad(xph, ((0, 0), (0, 0), (0, 0), (0, TP - T)))

    w2c = w2[0:C2, 0:3 * KH * C1].astype(jnp.bfloat16)
    w16 = w16.astype(jnp.bfloat16)

    conv_body = _make_conv_kernel(bb, T, TP)
    pa = pl.pallas_call(
        conv_body,
        out_shape=jax.ShapeDtypeStruct((B, 8, TP), jnp.float32),
        grid=(B // bb,),
        in_specs=[
            pl.BlockSpec((bb, KH, NPH, TP), lambda g: (g, 0, 0, 0)),
            pl.BlockSpec((16 * C1, NPH), lambda g: (0, 0)),
            pl.BlockSpec((C1, 1), lambda g: (0, 0)),
            pl.BlockSpec((C2, 3 * KH * C1), lambda g: (0, 0)),
            pl.BlockSpec((C2, 1), lambda g: (0, 0)),
            pl.BlockSpec((C1 // 2, 8), lambda g: (0, 0)),
            pl.BlockSpec((8, 1), lambda g: (0, 0)),
        ],
        out_specs=pl.BlockSpec((bb, 8, TP), lambda g: (g, 0, 0)),
        scratch_shapes=[pltpu.VMEM((6 * KH * C1, TP), jnp.bfloat16)],
        compiler_params=pltpu.CompilerParams(
            dimension_semantics=("parallel",)),
    )(xph, w16, b1, w2c, b2, wih, gbias.T)

    pat = pa[:, :, :T].reshape(4, 128, 2, 4, T)
    fwd = pat[:, :, 0]
    bwd = pat[:, :, 1][..., ::-1]
    ga = jnp.stack([fwd, bwd], axis=0)
    ga = ga.transpose(4, 3, 0, 1, 2).reshape(T, 4, 8, 128)

    whh_arr = jnp.repeat(whh, 4, axis=0).T.reshape(4, 8, 1)
    lw_arr = jnp.concatenate(
        [jnp.tile(lw[:, 0:1], (1, 4)), jnp.tile(lw[::-1, 1:2], (1, 4))],
        axis=1).reshape(T, 8, 1)

    scan_body = _make_scan_kernel(T)
    acc = pl.pallas_call(
        scan_body,
        out_shape=jax.ShapeDtypeStruct((8, 128), jnp.float32),
    )(ga, whh_arr, lw_arr)

    return (acc[0:4] + acc[4:8] + lb[0, 0]).reshape(B)


def kernel(x, w16, b1, w2, b2, wih, whh, gbias, lw, lb):
    return _run(x, w16, b1, w2, b2, wih, whh, gbias, lw, lb)


# speedup vs baseline: 18.8663x; 1.0195x over previous
import functools

import jax
import jax.numpy as jnp
from jax import lax
from jax.experimental import pallas as pl
from jax.experimental.pallas import tpu as pltpu

C1 = 64
C2 = 32
KH = 3
NPH = 18
NPHP = 32


def _make_conv_kernel(bb, T, TP):
    K1 = KH * C1

    def body(xph_ref, w16_ref, b1_ref, w2c_ref, b2_ref, wih_ref, gbt_ref,
             out_ref, h1_ref):
        lane = lax.broadcasted_iota(jnp.int32, (C1, TP), 1)
        b1 = b1_ref[...]
        b2 = b2_ref[...]
        gbt = gbt_ref[...]
        w2c = w2c_ref[...]
        wih = wih_ref[...]
        zcol = jnp.zeros((C1, 1), jnp.bfloat16)

        for b in range(bb):
            for kh in range(KH):
                ph = []
                for d in range(4):
                    rd = jnp.dot(w16_ref[256 * d:256 * d + 256, :],
                                 xph_ref[b, kh],
                                 preferred_element_type=jnp.float32)
                    pp = rd.reshape(4, C1, TP)
                    ph.append(((jnp.maximum(jnp.maximum(pp[0], pp[1]),
                                            jnp.maximum(pp[2], pp[3])) + b1)
                               ).astype(jnp.bfloat16))
                p0 = jnp.where(lane < T, ph[0], 0.0)
                row = kh * C1
                h1_ref[1 * K1 + row:1 * K1 + row + C1, :] = p0
                h1_ref[2 * K1 + row:2 * K1 + row + C1, :] = ph[1]
                h1_ref[3 * K1 + row:3 * K1 + row + C1, :] = ph[2]
                h1_ref[4 * K1 + row:4 * K1 + row + C1, :] = ph[3]
                h1_ref[0 * K1 + row:0 * K1 + row + C1, :] = jnp.concatenate(
                    [zcol, ph[3][:, :TP - 1]], axis=1)
                h1_ref[5 * K1 + row:5 * K1 + row + C1, :] = jnp.concatenate(
                    [p0[:, 1:], zcol], axis=1)

            f0 = jnp.dot(w2c, h1_ref[0 * K1:3 * K1, :],
                         preferred_element_type=jnp.float32)
            f1 = jnp.dot(w2c, h1_ref[1 * K1:4 * K1, :],
                         preferred_element_type=jnp.float32)
            f2 = jnp.dot(w2c, h1_ref[2 * K1:5 * K1, :],
                         preferred_element_type=jnp.float32)
            f3 = jnp.dot(w2c, h1_ref[3 * K1:6 * K1, :],
                         preferred_element_type=jnp.float32)
            feat = jnp.maximum(jnp.maximum(f0, f1),
                               jnp.maximum(f2, f3)) + b2

            proj = lax.dot_general(
                wih, feat, dimension_numbers=(((0,), (0,)), ((), ())),
                preferred_element_type=jnp.float32)
            out_ref[b] = proj + gbt

    return body


def _make_scan_kernel(T):
    def body(ga_ref, whh_ref, lw_ref, out_ref):
        whh = whh_ref[...]

        def step(t, carry):
            h, c, acc = carry
            gt = ga_ref[t]
            ti = jnp.tanh(gt[0] + h * whh[0])
            tf = jnp.tanh(gt[1] + h * whh[1])
            tg = jnp.tanh(gt[2] + h * whh[2])
            to = jnp.tanh(gt[3] + h * whh[3])
            i_s = ti * 0.5 + 0.5
            f_s = tf * 0.5 + 0.5
            o_s = to * 0.5 + 0.5
            c = f_s * c + i_s * tg
            h = o_s * jnp.tanh(c)
            acc = acc + h * lw_ref[t]
            return h, c, acc

        z = jnp.zeros((8, 128), jnp.float32)
        _, _, acc = lax.fori_loop(0, T, step, (z, z, z))
        out_ref[...] = acc

    return body


@jax.jit
def _run(x, w16, b1, w2, b2, wih, whh, gbias, lw, lb):
    B, H, W = x.shape
    T = lw.shape[0]
    TP = ((T + 127) // 128) * 128
    bb = 8
    assert H == KH and W == 16 * T and B % bb == 0 and B % 512 == 0

    xpad = jnp.pad(x.astype(jnp.bfloat16),
                   ((0, 0), (0, 0), (1, 16 * (T + 1) - W - 1)))
    x16 = xpad.reshape(B, KH, T + 1, 16).transpose(0, 1, 3, 2)
    xph = jnp.concatenate([x16[:, :, :, :T], x16[:, :, 0:2, 1:T + 1]], axis=2)
    xph = jnp.pad(xph, ((0, 0), (0, 0), (0, NPHP - NPH), (0, TP - T)))

    w2c = w2[0:C2, 0:3 * KH * C1].astype(jnp.bfloat16)
    w16 = jnp.pad(w16.astype(jnp.bfloat16), ((0, 0), (0, NPHP - NPH)))

    conv_body = _make_conv_kernel(bb, T, TP)
    pa = pl.pallas_call(
        conv_body,
        out_shape=jax.ShapeDtypeStruct((B, 8, TP), jnp.float32),
        grid=(B // bb,),
        in_specs=[
            pl.BlockSpec((bb, KH, NPHP, TP), lambda g: (g, 0, 0, 0)),
            pl.BlockSpec((16 * C1, NPHP), lambda g: (0, 0)),
            pl.BlockSpec((C1, 1), lambda g: (0, 0)),
            pl.BlockSpec((C2, 3 * KH * C1), lambda g: (0, 0)),
            pl.BlockSpec((C2, 1), lambda g: (0, 0)),
            pl.BlockSpec((C1 // 2, 8), lambda g: (0, 0)),
            pl.BlockSpec((8, 1), lambda g: (0, 0)),
        ],
        out_specs=pl.BlockSpec((bb, 8, TP), lambda g: (g, 0, 0)),
        scratch_shapes=[pltpu.VMEM((6 * KH * C1, TP), jnp.bfloat16)],
        compiler_params=pltpu.CompilerParams(
            dimension_semantics=("parallel",)),
    )(xph, w16, b1, w2c, b2, wih, gbias.T)

    pat = pa[:, :, :T].reshape(4, 128, 2, 4, T)
    fwd = pat[:, :, 0]
    bwd = pat[:, :, 1][..., ::-1]
    ga = jnp.stack([fwd, bwd], axis=0)
    ga = ga.transpose(4, 3, 0, 1, 2).reshape(T, 4, 8, 128)

    whh_arr = jnp.repeat(whh, 4, axis=0).T.reshape(4, 8, 1)
    lw_arr = jnp.concatenate(
        [jnp.tile(lw[:, 0:1], (1, 4)), jnp.tile(lw[::-1, 1:2], (1, 4))],
        axis=1).reshape(T, 8, 1)

    scan_body = _make_scan_kernel(T)
    acc = pl.pallas_call(
        scan_body,
        out_shape=jax.ShapeDtypeStruct((8, 128), jnp.float32),
    )(ga, whh_arr, lw_arr)

    return (acc[0:4] + acc[4:8] + lb[0, 0]).reshape(B)


def kernel(x, w16, b1, w2, b2, wih, whh, gbias, lw, lb):
    return _run(x, w16, b1, w2, b2, wih, whh, gbias, lw, lb)


# speedup vs baseline: 21.1760x; 1.1224x over previous
import functools

import jax
import jax.numpy as jnp
from jax import lax
from jax.experimental import pallas as pl
from jax.experimental.pallas import tpu as pltpu

C1 = 64
C2 = 32
KH = 3
NPH = 18
NPHP = 32


def _make_conv_kernel(bb, T, TP):
    K1 = KH * C1

    def body(xph_ref, w16_ref, b2_ref, w2c_ref, wih_ref, gbt_ref,
             out_ref, h1_ref):
        lane = lax.broadcasted_iota(jnp.int32, (C1, TP), 1)
        b2 = b2_ref[...]
        gbt = gbt_ref[...]
        w2c = w2c_ref[...]
        wih = wih_ref[...]
        zcol = jnp.zeros((C1, 1), jnp.bfloat16)

        for b in range(bb):
            for kh in range(KH):
                ph = []
                for d in range(4):
                    rd = jnp.dot(w16_ref[256 * d:256 * d + 256, :],
                                 xph_ref[b, kh],
                                 preferred_element_type=jnp.float32)
                    pp = rd.reshape(4, C1, TP)
                    ph.append((jnp.maximum(jnp.maximum(pp[0], pp[1]),
                                           jnp.maximum(pp[2], pp[3]))
                               ).astype(jnp.bfloat16))
                p0 = jnp.where(lane < T, ph[0], 0.0)
                row = kh * C1
                h1_ref[b, 1 * K1 + row:1 * K1 + row + C1, :] = p0
                h1_ref[b, 2 * K1 + row:2 * K1 + row + C1, :] = ph[1]
                h1_ref[b, 3 * K1 + row:3 * K1 + row + C1, :] = ph[2]
                h1_ref[b, 4 * K1 + row:4 * K1 + row + C1, :] = ph[3]
                h1_ref[b, 0 * K1 + row:0 * K1 + row + C1, :] = jnp.concatenate(
                    [zcol, ph[3][:, :TP - 1]], axis=1)
                h1_ref[b, 5 * K1 + row:5 * K1 + row + C1, :] = jnp.concatenate(
                    [p0[:, 1:], zcol], axis=1)

        for b in range(bb):
            f0 = jnp.dot(w2c, h1_ref[b, 0 * K1:3 * K1, :],
                         preferred_element_type=jnp.float32)
            f1 = jnp.dot(w2c, h1_ref[b, 1 * K1:4 * K1, :],
                         preferred_element_type=jnp.float32)
            f2 = jnp.dot(w2c, h1_ref[b, 2 * K1:5 * K1, :],
                         preferred_element_type=jnp.float32)
            f3 = jnp.dot(w2c, h1_ref[b, 3 * K1:6 * K1, :],
                         preferred_element_type=jnp.float32)
            feat = jnp.maximum(jnp.maximum(f0, f1),
                               jnp.maximum(f2, f3)) + b2

            proj = lax.dot_general(
                wih, feat, dimension_numbers=(((0,), (0,)), ((), ())),
                preferred_element_type=jnp.float32)
            out_ref[b] = proj + gbt

    return body


def _make_scan_kernel(T):
    def body(ga_ref, whh_ref, lw_ref, out_ref):
        whh = whh_ref[...]

        def step(t, carry):
            h, c, acc = carry
            gt = ga_ref[t]
            ti = jnp.tanh(gt[0] + h * whh[0])
            tf = jnp.tanh(gt[1] + h * whh[1])
            tg = jnp.tanh(gt[2] + h * whh[2])
            to = jnp.tanh(gt[3] + h * whh[3])
            i_s = ti * 0.5 + 0.5
            f_s = tf * 0.5 + 0.5
            o_s = to * 0.5 + 0.5
            c = f_s * c + i_s * tg
            h = o_s * jnp.tanh(c)
            acc = acc + h * lw_ref[t]
            return h, c, acc

        z = jnp.zeros((8, 128), jnp.float32)
        _, _, acc = lax.fori_loop(0, T, step, (z, z, z))
        out_ref[...] = acc

    return body


@jax.jit
def _run(x, w16, b1, w2, b2, wih, whh, gbias, lw, lb):
    B, H, W = x.shape
    T = lw.shape[0]
    TP = ((T + 127) // 128) * 128
    bb = 8
    assert H == KH and W == 16 * T and B % bb == 0 and B % 512 == 0

    xpad = jnp.pad(x.astype(jnp.bfloat16),
                   ((0, 0), (0, 0), (1, 16 * (T + 1) - W - 1)))
    x16 = xpad.reshape(B, KH, T + 1, 16).transpose(0, 1, 3, 2)
    ones = jnp.ones((B, KH, 1, T), jnp.bfloat16)
    xph = jnp.concatenate(
        [x16[:, :, :, :T], x16[:, :, 0:2, 1:T + 1], ones], axis=2)
    xph = jnp.pad(xph, ((0, 0), (0, 0), (0, NPHP - NPH - 1), (0, TP - T)))

    w2c = w2[0:C2, 0:3 * KH * C1].astype(jnp.bfloat16)
    w16 = jnp.concatenate(
        [w16, jnp.tile(b1, (16, 1))], axis=1).astype(jnp.bfloat16)
    w16 = jnp.pad(w16, ((0, 0), (0, NPHP - NPH - 1)))

    conv_body = _make_conv_kernel(bb, T, TP)
    pa = pl.pallas_call(
        conv_body,
        out_shape=jax.ShapeDtypeStruct((B, 8, TP), jnp.float32),
        grid=(B // bb,),
        in_specs=[
            pl.BlockSpec((bb, KH, NPHP, TP), lambda g: (g, 0, 0, 0)),
            pl.BlockSpec((16 * C1, NPHP), lambda g: (0, 0)),
            pl.BlockSpec((C2, 1), lambda g: (0, 0)),
            pl.BlockSpec((C2, 3 * KH * C1), lambda g: (0, 0)),
            pl.BlockSpec((C1 // 2, 8), lambda g: (0, 0)),
            pl.BlockSpec((8, 1), lambda g: (0, 0)),
        ],
        out_specs=pl.BlockSpec((bb, 8, TP), lambda g: (g, 0, 0)),
        scratch_shapes=[pltpu.VMEM((bb, 6 * KH * C1, TP), jnp.bfloat16)],
        compiler_params=pltpu.CompilerParams(
            dimension_semantics=("parallel",)),
    )(xph, w16, b2, w2c, wih, gbias.T)

    pat = pa[:, :, :T].reshape(4, 128, 2, 4, T)
    fwd = pat[:, :, 0]
    bwd = pat[:, :, 1][..., ::-1]
    ga = jnp.stack([fwd, bwd], axis=0)
    ga = ga.transpose(4, 3, 0, 1, 2).reshape(T, 4, 8, 128)

    whh_arr = jnp.repeat(whh, 4, axis=0).T.reshape(4, 8, 1)
    lw_arr = jnp.concatenate(
        [jnp.tile(lw[:, 0:1], (1, 4)), jnp.tile(lw[::-1, 1:2], (1, 4))],
        axis=1).reshape(T, 8, 1)

    scan_body = _make_scan_kernel(T)
    acc = pl.pallas_call(
        scan_body,
        out_shape=jax.ShapeDtypeStruct((8, 128), jnp.float32),
    )(ga, whh_arr, lw_arr)

    return (acc[0:4] + acc[4:8] + lb[0, 0]).reshape(B)


def kernel(x, w16, b1, w2, b2, wih, whh, gbias, lw, lb):
    return _run(x, w16, b1, w2, b2, wih, whh, gbias, lw, lb)


# speedup vs baseline: 21.2826x; 1.0050x over previous
import functools

import jax
import jax.numpy as jnp
from jax import lax
from jax.experimental import pallas as pl
from jax.experimental.pallas import tpu as pltpu

C1 = 64
C2 = 32
KH = 3
NPH = 18
NPHP = 32


def _make_conv_kernel(bb, T, TP):
    K1 = KH * C1

    def body(xph_ref, w16_ref, b2_ref, w2c_ref, wih_ref, gbt_ref,
             out_ref, h1_ref):
        lane = lax.broadcasted_iota(jnp.int32, (C1, TP), 1)
        b2 = b2_ref[...]
        gbt = gbt_ref[...]
        w2c = w2c_ref[...]
        wih = wih_ref[...]
        zcol = jnp.zeros((C1, 1), jnp.bfloat16)

        for b in range(bb):
            for kh in range(KH):
                ph = []
                for d in range(4):
                    rd = jnp.dot(w16_ref[256 * d:256 * d + 256, :],
                                 xph_ref[b, kh],
                                 preferred_element_type=jnp.float32)
                    pp = rd.reshape(4, C1, TP)
                    ph.append((jnp.maximum(jnp.maximum(pp[0], pp[1]),
                                           jnp.maximum(pp[2], pp[3]))
                               ).astype(jnp.bfloat16))
                p0 = jnp.where(lane < T, ph[0], 0.0)
                row = kh * C1
                h1_ref[b, 1 * K1 + row:1 * K1 + row + C1, :] = p0
                h1_ref[b, 2 * K1 + row:2 * K1 + row + C1, :] = ph[1]
                h1_ref[b, 3 * K1 + row:3 * K1 + row + C1, :] = ph[2]
                h1_ref[b, 4 * K1 + row:4 * K1 + row + C1, :] = ph[3]
                h1_ref[b, 0 * K1 + row:0 * K1 + row + C1, :] = jnp.concatenate(
                    [zcol, ph[3][:, :TP - 1]], axis=1)
                h1_ref[b, 5 * K1 + row:5 * K1 + row + C1, :] = jnp.concatenate(
                    [p0[:, 1:], zcol], axis=1)

        for b in range(bb):
            f0 = jnp.dot(w2c, h1_ref[b, 0 * K1:3 * K1, :],
                         preferred_element_type=jnp.float32)
            f1 = jnp.dot(w2c, h1_ref[b, 1 * K1:4 * K1, :],
                         preferred_element_type=jnp.float32)
            f2 = jnp.dot(w2c, h1_ref[b, 2 * K1:5 * K1, :],
                         preferred_element_type=jnp.float32)
            f3 = jnp.dot(w2c, h1_ref[b, 3 * K1:6 * K1, :],
                         preferred_element_type=jnp.float32)
            feat = jnp.maximum(jnp.maximum(f0, f1),
                               jnp.maximum(f2, f3)) + b2

            proj = lax.dot_general(
                wih, feat, dimension_numbers=(((0,), (0,)), ((), ())),
                preferred_element_type=jnp.float32)
            out_ref[b] = proj + gbt

    return body


def _make_scan_kernel(T):
    def body(ga_ref, whh_ref, lw_ref, out_ref):
        whh = whh_ref[...]

        def step(t, carry):
            h, c, acc = carry
            gt = ga_ref[t]
            ti = jnp.tanh(gt[0] + h * whh[0])
            tf = jnp.tanh(gt[1] + h * whh[1])
            tg = jnp.tanh(gt[2] + h * whh[2])
            to = jnp.tanh(gt[3] + h * whh[3])
            i_s = ti * 0.5 + 0.5
            f_s = tf * 0.5 + 0.5
            o_s = to * 0.5 + 0.5
            c = f_s * c + i_s * tg
            h = o_s * jnp.tanh(c)
            acc = acc + h * lw_ref[t]
            return h, c, acc

        z = jnp.zeros((8, 128), jnp.float32)
        _, _, acc = lax.fori_loop(0, T, step, (z, z, z))
        out_ref[...] = acc

    return body


@jax.jit
def _run(x, w16, b1, w2, b2, wih, whh, gbias, lw, lb):
    B, H, W = x.shape
    T = lw.shape[0]
    TP = ((T + 127) // 128) * 128
    bb = 16
    assert H == KH and W == 16 * T and B % bb == 0 and B % 512 == 0

    xpad = jnp.pad(x.astype(jnp.bfloat16),
                   ((0, 0), (0, 0), (1, 16 * (T + 1) - W - 1)))
    x16 = xpad.reshape(B, KH, T + 1, 16).transpose(0, 1, 3, 2)
    ones = jnp.ones((B, KH, 1, T), jnp.bfloat16)
    xph = jnp.concatenate(
        [x16[:, :, :, :T], x16[:, :, 0:2, 1:T + 1], ones], axis=2)
    xph = jnp.pad(xph, ((0, 0), (0, 0), (0, NPHP - NPH - 1), (0, TP - T)))

    w2c = w2[0:C2, 0:3 * KH * C1].astype(jnp.bfloat16)
    w16 = jnp.concatenate(
        [w16, jnp.tile(b1, (16, 1))], axis=1).astype(jnp.bfloat16)
    w16 = jnp.pad(w16, ((0, 0), (0, NPHP - NPH - 1)))

    conv_body = _make_conv_kernel(bb, T, TP)
    pa = pl.pallas_call(
        conv_body,
        out_shape=jax.ShapeDtypeStruct((B, 8, TP), jnp.float32),
        grid=(B // bb,),
        in_specs=[
            pl.BlockSpec((bb, KH, NPHP, TP), lambda g: (g, 0, 0, 0)),
            pl.BlockSpec((16 * C1, NPHP), lambda g: (0, 0)),
            pl.BlockSpec((C2, 1), lambda g: (0, 0)),
            pl.BlockSpec((C2, 3 * KH * C1), lambda g: (0, 0)),
            pl.BlockSpec((C1 // 2, 8), lambda g: (0, 0)),
            pl.BlockSpec((8, 1), lambda g: (0, 0)),
        ],
        out_specs=pl.BlockSpec((bb, 8, TP), lambda g: (g, 0, 0)),
        scratch_shapes=[pltpu.VMEM((bb, 6 * KH * C1, TP), jnp.bfloat16)],
        compiler_params=pltpu.CompilerParams(
            dimension_semantics=("parallel",)),
    )(xph, w16, b2, w2c, wih, gbias.T)

    pat = pa[:, :, :T].reshape(4, 128, 2, 4, T)
    fwd = pat[:, :, 0]
    bwd = pat[:, :, 1][..., ::-1]
    ga = jnp.stack([fwd, bwd], axis=0)
    ga = ga.transpose(4, 3, 0, 1, 2).reshape(T, 4, 8, 128)

    whh_arr = jnp.repeat(whh, 4, axis=0).T.reshape(4, 8, 1)
    lw_arr = jnp.concatenate(
        [jnp.tile(lw[:, 0:1], (1, 4)), jnp.tile(lw[::-1, 1:2], (1, 4))],
        axis=1).reshape(T, 8, 1)

    scan_body = _make_scan_kernel(T)
    acc = pl.pallas_call(
        scan_body,
        out_shape=jax.ShapeDtypeStruct((8, 128), jnp.float32),
    )(ga, whh_arr, lw_arr)

    return (acc[0:4] + acc[4:8] + lb[0, 0]).reshape(B)


def kernel(x, w16, b1, w2, b2, wih, whh, gbias, lw, lb):
    return _run(x, w16, b1, w2, b2, wih, whh, gbias, lw, lb)


# speedup vs baseline: 22.6239x; 1.0630x over previous
import functools

import jax
import jax.numpy as jnp
from jax import lax
from jax.experimental import pallas as pl
from jax.experimental.pallas import tpu as pltpu

C1 = 64
C2 = 32
KH = 3
NPH = 18
NPHP = 32


def _make_conv_kernel(bb, T, TP):
    K1 = KH * C1

    def body(xph_ref, w16_ref, b2_ref, w2c_ref, wih_ref, gbt_ref,
             out_ref, h1_ref):
        lane = lax.broadcasted_iota(jnp.int32, (C1, TP), 1)
        b2 = b2_ref[...]
        gbt = gbt_ref[...]
        w2c = w2c_ref[...]
        wih = wih_ref[...]
        zcol = jnp.zeros((C1, 1), jnp.bfloat16)

        for b in range(bb):
            for kh in range(KH):
                ph = []
                for d in range(4):
                    rd = jnp.dot(w16_ref[256 * d:256 * d + 256, :],
                                 xph_ref[b, kh],
                                 preferred_element_type=jnp.float32)
                    pp = rd.reshape(4, C1, TP)
                    ph.append((jnp.maximum(jnp.maximum(pp[0], pp[1]),
                                           jnp.maximum(pp[2], pp[3]))
                               ).astype(jnp.bfloat16))
                p0 = jnp.where(lane < T, ph[0], 0.0)
                row = kh * C1
                h1_ref[b, 1 * K1 + row:1 * K1 + row + C1, :] = p0
                h1_ref[b, 2 * K1 + row:2 * K1 + row + C1, :] = ph[1]
                h1_ref[b, 3 * K1 + row:3 * K1 + row + C1, :] = ph[2]
                h1_ref[b, 4 * K1 + row:4 * K1 + row + C1, :] = ph[3]
                h1_ref[b, 0 * K1 + row:0 * K1 + row + C1, :] = jnp.concatenate(
                    [zcol, ph[3][:, :TP - 1]], axis=1)
                h1_ref[b, 5 * K1 + row:5 * K1 + row + C1, :] = jnp.concatenate(
                    [p0[:, 1:], zcol], axis=1)

        for b in range(bb):
            s_all = jnp.dot(w2c, h1_ref[b],
                            preferred_element_type=jnp.float32)
            sb = s_all.reshape(4, C2, TP)
            feat = jnp.maximum(jnp.maximum(sb[0], sb[1]),
                               jnp.maximum(sb[2], sb[3])) + b2

            proj = lax.dot_general(
                wih, feat, dimension_numbers=(((0,), (0,)), ((), ())),
                preferred_element_type=jnp.float32)
            out_ref[b] = proj + gbt

    return body


def _make_scan_kernel(T):
    def body(ga_ref, whh_ref, lw_ref, out_ref):
        whh = whh_ref[...]

        def step(t, carry):
            h, c, acc = carry
            gt = ga_ref[t]
            ti = jnp.tanh(gt[0] + h * whh[0])
            tf = jnp.tanh(gt[1] + h * whh[1])
            tg = jnp.tanh(gt[2] + h * whh[2])
            to = jnp.tanh(gt[3] + h * whh[3])
            i_s = ti * 0.5 + 0.5
            f_s = tf * 0.5 + 0.5
            o_s = to * 0.5 + 0.5
            c = f_s * c + i_s * tg
            h = o_s * jnp.tanh(c)
            acc = acc + h * lw_ref[t]
            return h, c, acc

        z = jnp.zeros((8, 128), jnp.float32)
        _, _, acc = lax.fori_loop(0, T, step, (z, z, z))
        out_ref[...] = acc

    return body


@jax.jit
def _run(x, w16, b1, w2, b2, wih, whh, gbias, lw, lb):
    B, H, W = x.shape
    T = lw.shape[0]
    TP = ((T + 127) // 128) * 128
    bb = 16
    assert H == KH and W == 16 * T and B % bb == 0 and B % 512 == 0

    xpad = jnp.pad(x.astype(jnp.bfloat16),
                   ((0, 0), (0, 0), (1, 16 * (T + 1) - W - 1)))
    x16 = xpad.reshape(B, KH, T + 1, 16).transpose(0, 1, 3, 2)
    ones = jnp.ones((B, KH, 1, T), jnp.bfloat16)
    xph = jnp.concatenate(
        [x16[:, :, :, :T], x16[:, :, 0:2, 1:T + 1], ones], axis=2)
    xph = jnp.pad(xph, ((0, 0), (0, 0), (0, NPHP - NPH - 1), (0, TP - T)))

    w2c = w2.astype(jnp.bfloat16)
    w16 = jnp.concatenate(
        [w16, jnp.tile(b1, (16, 1))], axis=1).astype(jnp.bfloat16)
    w16 = jnp.pad(w16, ((0, 0), (0, NPHP - NPH - 1)))

    conv_body = _make_conv_kernel(bb, T, TP)
    pa = pl.pallas_call(
        conv_body,
        out_shape=jax.ShapeDtypeStruct((B, 8, TP), jnp.float32),
        grid=(B // bb,),
        in_specs=[
            pl.BlockSpec((bb, KH, NPHP, TP), lambda g: (g, 0, 0, 0)),
            pl.BlockSpec((16 * C1, NPHP), lambda g: (0, 0)),
            pl.BlockSpec((C2, 1), lambda g: (0, 0)),
            pl.BlockSpec((4 * C2, 6 * KH * C1), lambda g: (0, 0)),
            pl.BlockSpec((C1 // 2, 8), lambda g: (0, 0)),
            pl.BlockSpec((8, 1), lambda g: (0, 0)),
        ],
        out_specs=pl.BlockSpec((bb, 8, TP), lambda g: (g, 0, 0)),
        scratch_shapes=[pltpu.VMEM((bb, 6 * KH * C1, TP), jnp.bfloat16)],
        compiler_params=pltpu.CompilerParams(
            dimension_semantics=("parallel",)),
    )(xph, w16, b2, w2c, wih, gbias.T)

    pat = pa[:, :, :T].reshape(4, 128, 2, 4, T)
    fwd = pat[:, :, 0]
    bwd = pat[:, :, 1][..., ::-1]
    ga = jnp.stack([fwd, bwd], axis=0)
    ga = ga.transpose(4, 3, 0, 1, 2).reshape(T, 4, 8, 128)

    whh_arr = jnp.repeat(whh, 4, axis=0).T.reshape(4, 8, 1)
    lw_arr = jnp.concatenate(
        [jnp.tile(lw[:, 0:1], (1, 4)), jnp.tile(lw[::-1, 1:2], (1, 4))],
        axis=1).reshape(T, 8, 1)

    scan_body = _make_scan_kernel(T)
    acc = pl.pallas_call(
        scan_body,
        out_shape=jax.ShapeDtypeStruct((8, 128), jnp.float32),
    )(ga, whh_arr, lw_arr)

    return (acc[0:4] + acc[4:8] + lb[0, 0]).reshape(B)


def kernel(x, w16, b1, w2, b2, wih, whh, gbias, lw, lb):
    return _run(x, w16, b1, w2, b2, wih, whh, gbias, lw, lb)


# speedup vs baseline: 25.8124x; 1.1409x over previous
import functools

import jax
import jax.numpy as jnp
from jax import lax
from jax.experimental import pallas as pl
from jax.experimental.pallas import tpu as pltpu

C1 = 64
C2 = 32
KH = 3
NPH = 18
NPHP = 32


def _make_conv_kernel(bb, T, TP):
    K1 = KH * C1

    def body(xph_ref, w16_ref, b2_ref, w2c_ref, wih_ref, gbt_ref,
             out_ref, h1_ref):
        lane = lax.broadcasted_iota(jnp.int32, (C1, TP), 1)
        b2 = b2_ref[...]
        gbt = gbt_ref[...]
        w2c = w2c_ref[...]
        wih = wih_ref[...]
        zcol = jnp.zeros((C1, 1), jnp.bfloat16)

        for b in range(bb):
            for kh in range(KH):
                ph = []
                for d in range(4):
                    rd = jnp.dot(w16_ref[256 * d:256 * d + 256, :],
                                 xph_ref[b, kh],
                                 preferred_element_type=jnp.float32)
                    pp = rd.reshape(4, C1, TP)
                    ph.append((jnp.maximum(jnp.maximum(pp[0], pp[1]),
                                           jnp.maximum(pp[2], pp[3]))
                               ).astype(jnp.bfloat16))
                p0 = jnp.where(lane < T, ph[0], 0.0)
                row = kh * C1
                col = b * TP
                h1_ref[1 * K1 + row:1 * K1 + row + C1, col:col + TP] = p0
                h1_ref[2 * K1 + row:2 * K1 + row + C1, col:col + TP] = ph[1]
                h1_ref[3 * K1 + row:3 * K1 + row + C1, col:col + TP] = ph[2]
                h1_ref[4 * K1 + row:4 * K1 + row + C1, col:col + TP] = ph[3]
                h1_ref[0 * K1 + row:0 * K1 + row + C1,
                       col:col + TP] = jnp.concatenate(
                    [zcol, ph[3][:, :TP - 1]], axis=1)
                h1_ref[5 * K1 + row:5 * K1 + row + C1,
                       col:col + TP] = jnp.concatenate(
                    [p0[:, 1:], zcol], axis=1)

        s_all = jnp.dot(w2c, h1_ref[...],
                        preferred_element_type=jnp.float32)
        sb = s_all.reshape(4, C2, bb * TP)
        feat = jnp.maximum(jnp.maximum(sb[0], sb[1]),
                           jnp.maximum(sb[2], sb[3])) + b2

        proj = lax.dot_general(
            wih, feat, dimension_numbers=(((0,), (0,)), ((), ())),
            preferred_element_type=jnp.float32)
        for b in range(bb):
            out_ref[b] = proj[:, b * TP:(b + 1) * TP] + gbt

    return body


def _make_scan_kernel(T):
    def body(ga_ref, whh_ref, lw_ref, out_ref):
        whh = whh_ref[...]

        def step(t, carry):
            h, c, acc = carry
            gt = ga_ref[t]
            ti = jnp.tanh(gt[0] + h * whh[0])
            tf = jnp.tanh(gt[1] + h * whh[1])
            tg = jnp.tanh(gt[2] + h * whh[2])
            to = jnp.tanh(gt[3] + h * whh[3])
            i_s = ti * 0.5 + 0.5
            f_s = tf * 0.5 + 0.5
            o_s = to * 0.5 + 0.5
            c = f_s * c + i_s * tg
            h = o_s * jnp.tanh(c)
            acc = acc + h * lw_ref[t]
            return h, c, acc

        z = jnp.zeros((8, 128), jnp.float32)
        _, _, acc = lax.fori_loop(0, T, step, (z, z, z))
        out_ref[...] = acc

    return body


@jax.jit
def _run(x, w16, b1, w2, b2, wih, whh, gbias, lw, lb):
    B, H, W = x.shape
    T = lw.shape[0]
    TP = ((T + 127) // 128) * 128
    bb = 16
    assert H == KH and W == 16 * T and B % bb == 0 and B % 512 == 0

    xpad = jnp.pad(x.astype(jnp.bfloat16),
                   ((0, 0), (0, 0), (1, 16 * (T + 1) - W - 1)))
    x16 = xpad.reshape(B, KH, T + 1, 16).transpose(0, 1, 3, 2)
    ones = jnp.ones((B, KH, 1, T), jnp.bfloat16)
    xph = jnp.concatenate(
        [x16[:, :, :, :T], x16[:, :, 0:2, 1:T + 1], ones], axis=2)
    xph = jnp.pad(xph, ((0, 0), (0, 0), (0, NPHP - NPH - 1), (0, TP - T)))

    w2c = w2.astype(jnp.bfloat16)
    w16 = jnp.concatenate(
        [w16, jnp.tile(b1, (16, 1))], axis=1).astype(jnp.bfloat16)
    w16 = jnp.pad(w16, ((0, 0), (0, NPHP - NPH - 1)))

    conv_body = _make_conv_kernel(bb, T, TP)
    pa = pl.pallas_call(
        conv_body,
        out_shape=jax.ShapeDtypeStruct((B, 8, TP), jnp.float32),
        grid=(B // bb,),
        in_specs=[
            pl.BlockSpec((bb, KH, NPHP, TP), lambda g: (g, 0, 0, 0)),
            pl.BlockSpec((16 * C1, NPHP), lambda g: (0, 0)),
            pl.BlockSpec((C2, 1), lambda g: (0, 0)),
            pl.BlockSpec((4 * C2, 6 * KH * C1), lambda g: (0, 0)),
            pl.BlockSpec((C1 // 2, 8), lambda g: (0, 0)),
            pl.BlockSpec((8, 1), lambda g: (0, 0)),
        ],
        out_specs=pl.BlockSpec((bb, 8, TP), lambda g: (g, 0, 0)),
        scratch_shapes=[pltpu.VMEM((6 * KH * C1, bb * TP), jnp.bfloat16)],
        compiler_params=pltpu.CompilerParams(
            dimension_semantics=("parallel",)),
    )(xph, w16, b2, w2c, wih, gbias.T)

    pat = pa[:, :, :T].reshape(4, 128, 2, 4, T)
    fwd = pat[:, :, 0]
    bwd = pat[:, :, 1][..., ::-1]
    ga = jnp.stack([fwd, bwd], axis=0)
    ga = ga.transpose(4, 3, 0, 1, 2).reshape(T, 4, 8, 128)

    whh_arr = jnp.repeat(whh, 4, axis=0).T.reshape(4, 8, 1)
    lw_arr = jnp.concatenate(
        [jnp.tile(lw[:, 0:1], (1, 4)), jnp.tile(lw[::-1, 1:2], (1, 4))],
        axis=1).reshape(T, 8, 1)

    scan_body = _make_scan_kernel(T)
    acc = pl.pallas_call(
        scan_body,
        out_shape=jax.ShapeDtypeStruct((8, 128), jnp.float32),
    )(ga, whh_arr, lw_arr)

    return (acc[0:4] + acc[4:8] + lb[0, 0]).reshape(B)


def kernel(x, w16, b1, w2, b2, wih, whh, gbias, lw, lb):
    return _run(x, w16, b1, w2, b2, wih, whh, gbias, lw, lb)
